# baseline (speedup 1.0000x reference)
import jax
import jax.numpy as jnp
from jax.experimental import pallas as pl
from jax.experimental.pallas import tpu as pltpu

B, L, D = 256, 512, 768
BB = 8
N_OUT = 54


def _pool_project_body(h_ref, m_ref, wt_ref, b_ref, o_ref):
    h = h_ref[...]
    m = m_ref[...]
    s = jnp.sum(h * m[:, :, None], axis=1)
    cnt = jnp.sum(m, axis=1, keepdims=True)
    pooled = s / cnt
    o_ref[...] = (
        jnp.dot(pooled, wt_ref[...], preferred_element_type=jnp.float32)
        + b_ref[...]
    )


def kernel(final_hidden, attention_mask, W13, b13, W14, b14):
    w = jnp.concatenate([W13.reshape(13 * 4, D), W14], axis=0)
    wt = w.T
    b = jnp.concatenate([b13.reshape(13 * 4), b14])[None, :]
    mask = attention_mask.astype(jnp.float32)

    out = pl.pallas_call(
        _pool_project_body,
        out_shape=jax.ShapeDtypeStruct((B, N_OUT), jnp.float32),
        grid=(B // BB,),
        in_specs=[
            pl.BlockSpec((BB, L, D), lambda i: (i, 0, 0)),
            pl.BlockSpec((BB, L), lambda i: (i, 0)),
            pl.BlockSpec((D, N_OUT), lambda i: (0, 0)),
            pl.BlockSpec((1, N_OUT), lambda i: (0, 0)),
        ],
        out_specs=pl.BlockSpec((BB, N_OUT), lambda i: (i, 0)),
        compiler_params=pltpu.CompilerParams(
            dimension_semantics=("parallel",),
            vmem_limit_bytes=56 * 1024 * 1024,
        ),
        name="bow_labeler_pool_project",
    )(final_hidden, mask, wt, b)
    return out
